# baseline (speedup 1.0000x reference)
import functools
import math

import jax
import jax.numpy as jnp
from jax import lax
from jax.experimental import pallas as pl
from jax.experimental.pallas import tpu as pltpu
from jax.experimental.pallas import tpu_sc as plsc

D_MODEL = 768
LANES = 16
NC, NS = 2, 16
NW = NC * NS
CHUNK = 64
NBUF = 2
LOOKAHEAD = 1
SCALE = math.sqrt(float(D_MODEL))


def _scale_chunk(buf):
    def row_body(r, carry):
        for j in range(D_MODEL // LANES):
            sl = pl.ds(j * LANES, LANES)
            buf[r, sl] = buf[r, sl] * SCALE
        return carry

    lax.fori_loop(0, CHUNK, row_body, 0)


def _emb_body(n_tokens, ids_hbm, w_hbm, out_hbm, idx_v, *bufs_and_sems):
    bufs = bufs_and_sems[:NBUF]
    gsems = bufs_and_sems[NBUF:2 * NBUF]
    osems = bufs_and_sems[2 * NBUF:3 * NBUF]
    b_per_w = n_tokens // NW
    n_chunks = b_per_w // CHUNK
    wid = lax.axis_index("s") * NC + lax.axis_index("c")
    base = wid * b_per_w
    pltpu.sync_copy(ids_hbm.at[pl.ds(base, b_per_w)], idx_v)

    def start_gather(h):
        pltpu.async_copy(
            w_hbm.at[idx_v.at[pl.ds(h * CHUNK, CHUNK)]],
            bufs[h % NBUF],
            gsems[h % NBUF],
        )

    def gather_done(h):
        pltpu.make_async_copy(
            w_hbm.at[idx_v.at[pl.ds(h * CHUNK, CHUNK)]],
            bufs[h % NBUF],
            gsems[h % NBUF],
        ).wait()

    def start_out(g):
        pltpu.async_copy(
            bufs[g % NBUF],
            out_hbm.at[pl.ds(base + g * CHUNK, CHUNK)],
            osems[g % NBUF],
        )

    def out_done(g):
        pltpu.make_async_copy(
            bufs[g % NBUF],
            out_hbm.at[pl.ds(base + g * CHUNK, CHUNK)],
            osems[g % NBUF],
        ).wait()

    outs_waited = set()
    for h in range(LOOKAHEAD):
        start_gather(h)
    for g in range(n_chunks):
        b = g % NBUF
        gather_done(g)
        _scale_chunk(bufs[b])
        start_out(g)
        h = g + LOOKAHEAD
        if h < n_chunks:
            if h >= NBUF:
                out_done(h - NBUF)
                outs_waited.add(h - NBUF)
            start_gather(h)
    for g in range(n_chunks):
        if g not in outs_waited:
            out_done(g)


@functools.partial(jax.jit, static_argnames=())
def _emb_lookup(ids_flat, W):
    n_tokens = ids_flat.shape[0]
    mesh = plsc.VectorSubcoreMesh(core_axis_name="c", subcore_axis_name="s")
    body = functools.partial(_emb_body, n_tokens)
    scratch = [pltpu.VMEM((n_tokens // NW,), jnp.int32)]
    scratch += [pltpu.VMEM((CHUNK, D_MODEL), jnp.float32) for _ in range(NBUF)]
    scratch += [pltpu.SemaphoreType.DMA for _ in range(2 * NBUF)]
    run = pl.kernel(
        body,
        out_type=jax.ShapeDtypeStruct((n_tokens, D_MODEL), jnp.float32),
        mesh=mesh,
        scratch_types=scratch,
    )
    return run(ids_flat, W)


def kernel(input_ids, W):
    b, l = input_ids.shape
    out = _emb_lookup(input_ids.reshape(b * l), W)
    return out.reshape(b, l, D_MODEL)


# speedup vs baseline: 1.2343x; 1.2343x over previous
import functools
import math

import jax
import jax.numpy as jnp
from jax import lax
from jax.experimental import pallas as pl
from jax.experimental.pallas import tpu as pltpu
from jax.experimental.pallas import tpu_sc as plsc

D_MODEL = 768
LANES = 16
NC, NS = 2, 16
NW = NC * NS
CHUNK = 16
NBUF = 8
LOOKAHEAD = 4
SCALE = math.sqrt(float(D_MODEL))


def _scale_chunk(buf):
    def row_body(r, carry):
        for j in range(D_MODEL // LANES):
            sl = pl.ds(j * LANES, LANES)
            buf[r, sl] = buf[r, sl] * SCALE
        return carry

    lax.fori_loop(0, CHUNK, row_body, 0)


def _emb_body(n_tokens, ids_hbm, w_hbm, out_hbm, idx_v, *bufs_and_sems):
    bufs = bufs_and_sems[:NBUF]
    gsems = bufs_and_sems[NBUF:2 * NBUF]
    osems = bufs_and_sems[2 * NBUF:3 * NBUF]
    b_per_w = n_tokens // NW
    n_chunks = b_per_w // CHUNK
    n_groups = n_chunks // NBUF
    assert n_groups * NBUF == n_chunks and n_groups >= 3
    wid = lax.axis_index("s") * NC + lax.axis_index("c")
    base = wid * b_per_w
    pltpu.sync_copy(ids_hbm.at[pl.ds(base, b_per_w)], idx_v)

    def start_gather(c, b):
        pltpu.async_copy(
            w_hbm.at[idx_v.at[pl.ds(c * CHUNK, CHUNK)]], bufs[b], gsems[b]
        )

    def gather_done(c, b):
        pltpu.make_async_copy(
            w_hbm.at[idx_v.at[pl.ds(c * CHUNK, CHUNK)]], bufs[b], gsems[b]
        ).wait()

    def start_out(c, b):
        pltpu.async_copy(
            bufs[b], out_hbm.at[pl.ds(base + c * CHUNK, CHUNK)], osems[b]
        )

    def out_done(c, b):
        pltpu.make_async_copy(
            bufs[b], out_hbm.at[pl.ds(base + c * CHUNK, CHUNK)], osems[b]
        ).wait()

    def step(c, b):
        gather_done(c, b)
        _scale_chunk(bufs[b])
        start_out(c, b)
        h = c + LOOKAHEAD
        hb = (b + LOOKAHEAD) % NBUF
        out_done(h - NBUF, hb)
        start_gather(h, hb)

    for b in range(LOOKAHEAD):
        start_gather(b, b)
    for b in range(NBUF):
        gather_done(b, b)
        _scale_chunk(bufs[b])
        start_out(b, b)
        h = b + LOOKAHEAD
        hb = (b + LOOKAHEAD) % NBUF
        if h >= NBUF:
            out_done(h - NBUF, hb)
        start_gather(h, hb)

    def group_body(g, carry):
        c0 = g * NBUF
        for b in range(NBUF):
            step(c0 + b, b)
        return carry

    lax.fori_loop(1, n_groups - 1, group_body, 0)

    c0 = (n_groups - 1) * NBUF
    for b in range(NBUF):
        gather_done(c0 + b, b)
        _scale_chunk(bufs[b])
        start_out(c0 + b, b)
        h = c0 + b + LOOKAHEAD
        hb = (b + LOOKAHEAD) % NBUF
        if h < n_chunks:
            out_done(h - NBUF, hb)
            start_gather(h, hb)
    for c in range(c0, n_chunks):
        out_done(c, c % NBUF)


@functools.partial(jax.jit, static_argnames=())
def _emb_lookup(ids_flat, W):
    n_tokens = ids_flat.shape[0]
    mesh = plsc.VectorSubcoreMesh(core_axis_name="c", subcore_axis_name="s")
    body = functools.partial(_emb_body, n_tokens)
    scratch = [pltpu.VMEM((n_tokens // NW,), jnp.int32)]
    scratch += [pltpu.VMEM((CHUNK, D_MODEL), jnp.float32) for _ in range(NBUF)]
    scratch += [pltpu.SemaphoreType.DMA for _ in range(2 * NBUF)]
    run = pl.kernel(
        body,
        out_type=jax.ShapeDtypeStruct((n_tokens, D_MODEL), jnp.float32),
        mesh=mesh,
        scratch_types=scratch,
    )
    return run(ids_flat, W)


def kernel(input_ids, W):
    b, l = input_ids.shape
    out = _emb_lookup(input_ids.reshape(b * l), W)
    return out.reshape(b, l, D_MODEL)


# speedup vs baseline: 1.2397x; 1.0044x over previous
import functools
import math

import jax
import jax.numpy as jnp
from jax import lax
from jax.experimental import pallas as pl
from jax.experimental.pallas import tpu as pltpu
from jax.experimental.pallas import tpu_sc as plsc

D_MODEL = 768
LANES = 16
NC, NS = 2, 16
NW = NC * NS
CHUNK = 16
NBUF = 8
LOOKAHEAD = 4
SCALE = math.sqrt(float(D_MODEL))


def _scale_chunk(buf):
    def row_body(r, carry):
        for j in range(D_MODEL // LANES):
            sl = pl.ds(j * LANES, LANES)
            buf[r, sl] = buf[r, sl] * SCALE
        return carry

    lax.fori_loop(0, CHUNK, row_body, 0)


def _emb_body(n_tokens, ids_hbm, w_hbm, out_hbm, idx_v, *bufs_and_sems):
    bufs = bufs_and_sems[:NBUF]
    gsems = bufs_and_sems[NBUF:2 * NBUF]
    osems = bufs_and_sems[2 * NBUF:3 * NBUF]
    b_per_w = n_tokens // NW
    n_chunks = b_per_w // CHUNK
    n_groups = n_chunks // NBUF
    assert n_groups * NBUF == n_chunks and n_groups >= 3
    wid = lax.axis_index("s") * NC + lax.axis_index("c")
    base = wid * b_per_w
    pltpu.sync_copy(ids_hbm.at[pl.ds(base, b_per_w)], idx_v)

    def start_gather(c, b):
        pltpu.async_copy(
            w_hbm.at[idx_v.at[pl.ds(c * CHUNK, CHUNK)]], bufs[b], gsems[b]
        )

    def gather_done(c, b):
        pltpu.make_async_copy(
            w_hbm.at[idx_v.at[pl.ds(c * CHUNK, CHUNK)]], bufs[b], gsems[b]
        ).wait()

    def start_out(c, b):
        pltpu.async_copy(
            bufs[b], out_hbm.at[pl.ds(base + c * CHUNK, CHUNK)], osems[b]
        )

    def out_done(c, b):
        pltpu.make_async_copy(
            bufs[b], out_hbm.at[pl.ds(base + c * CHUNK, CHUNK)], osems[b]
        ).wait()

    def step(c, b):
        gather_done(c, b)
        h = c + LOOKAHEAD
        hb = (b + LOOKAHEAD) % NBUF
        out_done(h - NBUF, hb)
        start_gather(h, hb)
        _scale_chunk(bufs[b])
        start_out(c, b)

    for b in range(LOOKAHEAD):
        start_gather(b, b)
    for b in range(NBUF):
        gather_done(b, b)
        _scale_chunk(bufs[b])
        start_out(b, b)
        h = b + LOOKAHEAD
        hb = (b + LOOKAHEAD) % NBUF
        if h >= NBUF:
            out_done(h - NBUF, hb)
        start_gather(h, hb)

    def group_body(g, carry):
        c0 = g * NBUF
        for b in range(NBUF):
            step(c0 + b, b)
        return carry

    lax.fori_loop(1, n_groups - 1, group_body, 0)

    c0 = (n_groups - 1) * NBUF
    for b in range(NBUF):
        gather_done(c0 + b, b)
        _scale_chunk(bufs[b])
        start_out(c0 + b, b)
        h = c0 + b + LOOKAHEAD
        hb = (b + LOOKAHEAD) % NBUF
        if h < n_chunks:
            out_done(h - NBUF, hb)
            start_gather(h, hb)
    for c in range(c0, n_chunks):
        out_done(c, c % NBUF)


@functools.partial(jax.jit, static_argnames=())
def _emb_lookup(ids_flat, W):
    n_tokens = ids_flat.shape[0]
    mesh = plsc.VectorSubcoreMesh(core_axis_name="c", subcore_axis_name="s")
    body = functools.partial(_emb_body, n_tokens)
    scratch = [pltpu.VMEM((n_tokens // NW,), jnp.int32)]
    scratch += [pltpu.VMEM((CHUNK, D_MODEL), jnp.float32) for _ in range(NBUF)]
    scratch += [pltpu.SemaphoreType.DMA for _ in range(2 * NBUF)]
    run = pl.kernel(
        body,
        out_type=jax.ShapeDtypeStruct((n_tokens, D_MODEL), jnp.float32),
        mesh=mesh,
        scratch_types=scratch,
    )
    return run(ids_flat, W)


def kernel(input_ids, W):
    b, l = input_ids.shape
    out = _emb_lookup(input_ids.reshape(b * l), W)
    return out.reshape(b, l, D_MODEL)


# speedup vs baseline: 1.2461x; 1.0052x over previous
import functools
import math

import jax
import jax.numpy as jnp
from jax import lax
from jax.experimental import pallas as pl
from jax.experimental.pallas import tpu as pltpu
from jax.experimental.pallas import tpu_sc as plsc

D_MODEL = 768
LANES = 16
NC, NS = 2, 16
NW = NC * NS
CHUNK = 16
NBUF = 8
LOOKAHEAD = 4
SCALE = math.sqrt(float(D_MODEL))


def _scale_chunk(buf):
    def row_body(r, carry):
        for j in range(D_MODEL // LANES):
            sl = pl.ds(j * LANES, LANES)
            buf[r, sl] = buf[r, sl] * SCALE
        return carry

    lax.fori_loop(0, CHUNK, row_body, 0)


def _emb_body(batch, seq, ids_hbm, w_hbm, out_hbm, idx_v, *bufs_and_sems):
    bufs = bufs_and_sems[:NBUF]
    gsems = bufs_and_sems[NBUF:2 * NBUF]
    osems = bufs_and_sems[2 * NBUF:3 * NBUF]
    b_per_w = (batch * seq) // NW
    n_chunks = b_per_w // CHUNK
    n_groups = n_chunks // NBUF
    assert n_groups * NBUF == n_chunks and n_groups >= 3
    assert seq % b_per_w == 0
    wid = lax.axis_index("s") * NC + lax.axis_index("c")
    base = wid * b_per_w
    row = base // seq
    col = base % seq
    pltpu.sync_copy(ids_hbm.at[row, pl.ds(col, b_per_w)], idx_v)

    def start_gather(c, b):
        pltpu.async_copy(
            w_hbm.at[idx_v.at[pl.ds(c * CHUNK, CHUNK)]], bufs[b], gsems[b]
        )

    def gather_done(c, b):
        pltpu.make_async_copy(
            w_hbm.at[idx_v.at[pl.ds(c * CHUNK, CHUNK)]], bufs[b], gsems[b]
        ).wait()

    def start_out(c, b):
        pltpu.async_copy(
            bufs[b], out_hbm.at[row, pl.ds(col + c * CHUNK, CHUNK)], osems[b]
        )

    def out_done(c, b):
        pltpu.make_async_copy(
            bufs[b], out_hbm.at[row, pl.ds(col + c * CHUNK, CHUNK)], osems[b]
        ).wait()

    def step(c, b):
        gather_done(c, b)
        h = c + LOOKAHEAD
        hb = (b + LOOKAHEAD) % NBUF
        out_done(h - NBUF, hb)
        start_gather(h, hb)
        _scale_chunk(bufs[b])
        start_out(c, b)

    for b in range(LOOKAHEAD):
        start_gather(b, b)
    for b in range(NBUF):
        gather_done(b, b)
        h = b + LOOKAHEAD
        hb = (b + LOOKAHEAD) % NBUF
        if h >= NBUF:
            out_done(h - NBUF, hb)
        start_gather(h, hb)
        _scale_chunk(bufs[b])
        start_out(b, b)

    def group_body(g, carry):
        c0 = g * NBUF
        for b in range(NBUF):
            step(c0 + b, b)
        return carry

    lax.fori_loop(1, n_groups - 1, group_body, 0)

    c0 = (n_groups - 1) * NBUF
    for b in range(NBUF):
        gather_done(c0 + b, b)
        h = c0 + b + LOOKAHEAD
        hb = (b + LOOKAHEAD) % NBUF
        if h < n_chunks:
            out_done(h - NBUF, hb)
            start_gather(h, hb)
        _scale_chunk(bufs[b])
        start_out(c0 + b, b)
    for c in range(c0, n_chunks):
        out_done(c, c % NBUF)


@functools.partial(jax.jit, static_argnames=())
def _emb_lookup(ids, W):
    batch, seq = ids.shape
    mesh = plsc.VectorSubcoreMesh(core_axis_name="c", subcore_axis_name="s")
    body = functools.partial(_emb_body, batch, seq)
    scratch = [pltpu.VMEM(((batch * seq) // NW,), jnp.int32)]
    scratch += [pltpu.VMEM((CHUNK, D_MODEL), jnp.float32) for _ in range(NBUF)]
    scratch += [pltpu.SemaphoreType.DMA for _ in range(2 * NBUF)]
    run = pl.kernel(
        body,
        out_type=jax.ShapeDtypeStruct((batch, seq, D_MODEL), jnp.float32),
        mesh=mesh,
        scratch_types=scratch,
    )
    return run(ids, W)


def kernel(input_ids, W):
    return _emb_lookup(input_ids, W)


# speedup vs baseline: 1.2545x; 1.0067x over previous
import functools
import math

import jax
import jax.numpy as jnp
from jax import lax
from jax.experimental import pallas as pl
from jax.experimental.pallas import tpu as pltpu
from jax.experimental.pallas import tpu_sc as plsc

D_MODEL = 768
LANES = 16
NC, NS = 2, 16
NW = NC * NS
CHUNK = 16
NBUF = 8
LOOKAHEAD = 6
SCALE = math.sqrt(float(D_MODEL))


def _scale_chunk(buf):
    def row_body(r, carry):
        for j in range(D_MODEL // LANES):
            sl = pl.ds(j * LANES, LANES)
            buf[r, sl] = buf[r, sl] * SCALE
        return carry

    lax.fori_loop(0, CHUNK, row_body, 0)


def _emb_body(batch, seq, ids_hbm, w_hbm, out_hbm, idx_v, *bufs_and_sems):
    bufs = bufs_and_sems[:NBUF]
    gsems = bufs_and_sems[NBUF:2 * NBUF]
    osems = bufs_and_sems[2 * NBUF:3 * NBUF]
    b_per_w = (batch * seq) // NW
    n_chunks = b_per_w // CHUNK
    n_groups = n_chunks // NBUF
    assert n_groups * NBUF == n_chunks and n_groups >= 3
    assert seq % b_per_w == 0
    wid = lax.axis_index("s") * NC + lax.axis_index("c")
    base = wid * b_per_w
    row = base // seq
    col = base % seq
    pltpu.sync_copy(ids_hbm.at[row, pl.ds(col, b_per_w)], idx_v)

    def start_gather(c, b):
        pltpu.async_copy(
            w_hbm.at[idx_v.at[pl.ds(c * CHUNK, CHUNK)]], bufs[b], gsems[b]
        )

    def gather_done(c, b):
        pltpu.make_async_copy(
            w_hbm.at[idx_v.at[pl.ds(c * CHUNK, CHUNK)]], bufs[b], gsems[b]
        ).wait()

    def start_out(c, b):
        pltpu.async_copy(
            bufs[b], out_hbm.at[row, pl.ds(col + c * CHUNK, CHUNK)], osems[b]
        )

    def out_done(c, b):
        pltpu.make_async_copy(
            bufs[b], out_hbm.at[row, pl.ds(col + c * CHUNK, CHUNK)], osems[b]
        ).wait()

    def step(c, b):
        gather_done(c, b)
        h = c + LOOKAHEAD
        hb = (b + LOOKAHEAD) % NBUF
        out_done(h - NBUF, hb)
        start_gather(h, hb)
        _scale_chunk(bufs[b])
        start_out(c, b)

    for b in range(LOOKAHEAD):
        start_gather(b, b)
    for b in range(NBUF):
        gather_done(b, b)
        h = b + LOOKAHEAD
        hb = (b + LOOKAHEAD) % NBUF
        if h >= NBUF:
            out_done(h - NBUF, hb)
        start_gather(h, hb)
        _scale_chunk(bufs[b])
        start_out(b, b)

    def group_body(g, carry):
        c0 = g * NBUF
        for b in range(NBUF):
            step(c0 + b, b)
        return carry

    lax.fori_loop(1, n_groups - 1, group_body, 0)

    c0 = (n_groups - 1) * NBUF
    for b in range(NBUF):
        gather_done(c0 + b, b)
        h = c0 + b + LOOKAHEAD
        hb = (b + LOOKAHEAD) % NBUF
        if h < n_chunks:
            out_done(h - NBUF, hb)
            start_gather(h, hb)
        _scale_chunk(bufs[b])
        start_out(c0 + b, b)
    for c in range(c0, n_chunks):
        out_done(c, c % NBUF)


@functools.partial(jax.jit, static_argnames=())
def _emb_lookup(ids, W):
    batch, seq = ids.shape
    mesh = plsc.VectorSubcoreMesh(core_axis_name="c", subcore_axis_name="s")
    body = functools.partial(_emb_body, batch, seq)
    scratch = [pltpu.VMEM(((batch * seq) // NW,), jnp.int32)]
    scratch += [pltpu.VMEM((CHUNK, D_MODEL), jnp.float32) for _ in range(NBUF)]
    scratch += [pltpu.SemaphoreType.DMA for _ in range(2 * NBUF)]
    run = pl.kernel(
        body,
        out_type=jax.ShapeDtypeStruct((batch, seq, D_MODEL), jnp.float32),
        mesh=mesh,
        scratch_types=scratch,
    )
    return run(ids, W)


def kernel(input_ids, W):
    return _emb_lookup(input_ids, W)


# speedup vs baseline: 1.3208x; 1.0529x over previous
import functools
import math

import jax
import jax.numpy as jnp
from jax import lax
from jax.experimental import pallas as pl
from jax.experimental.pallas import tpu as pltpu
from jax.experimental.pallas import tpu_sc as plsc

D_MODEL = 768
LANES = 16
NC, NS = 2, 16
NW = NC * NS
CHUNK = 16
NBUF = 8
LOOKAHEAD = 6
SCALE = math.sqrt(float(D_MODEL))


def _scale_chunk(buf):
    def row_body(r, carry):
        for j in range(D_MODEL // LANES):
            sl = pl.ds(j * LANES, LANES)
            buf[r, sl] = buf[r, sl] * SCALE
        return carry

    lax.fori_loop(0, CHUNK, row_body, 0)


def _emb_body(batch, seq, ids_hbm, w_hbm, out_hbm, idx_v, *bufs_and_sems):
    bufs = bufs_and_sems[:NBUF]
    gsems = bufs_and_sems[NBUF:2 * NBUF]
    osems = bufs_and_sems[2 * NBUF:3 * NBUF]
    b_per_w = (batch * seq) // NW
    n_chunks = b_per_w // CHUNK
    n_groups = n_chunks // NBUF
    assert n_groups * NBUF == n_chunks and n_groups >= 3
    assert seq % b_per_w == 0
    wid = lax.axis_index("s") * NC + lax.axis_index("c")
    base = wid * b_per_w
    row = base // seq
    col = base % seq
    pltpu.sync_copy(ids_hbm.at[row, pl.ds(col, b_per_w)], idx_v)

    def start_gather(c, b):
        pltpu.async_copy(
            w_hbm.at[idx_v.at[pl.ds(c * CHUNK, CHUNK)]], bufs[b], gsems[b]
        )

    def gather_done(c, b):
        pltpu.make_async_copy(
            w_hbm.at[idx_v.at[pl.ds(c * CHUNK, CHUNK)]], bufs[b], gsems[b]
        ).wait()

    def start_out(c, b):
        pltpu.async_copy(
            bufs[b], out_hbm.at[row, pl.ds(col + c * CHUNK, CHUNK)], osems[b]
        )

    def out_done(c, b):
        pltpu.make_async_copy(
            bufs[b], out_hbm.at[row, pl.ds(col + c * CHUNK, CHUNK)], osems[b]
        ).wait()

    for b in range(LOOKAHEAD):
        start_gather(b, b)

    def group_body(g, carry):
        c0 = g * NBUF
        for b in range(NBUF):
            c = c0 + b
            gather_done(c, b)
            h = c + LOOKAHEAD
            hb = (b + LOOKAHEAD) % NBUF

            @pl.when(h - NBUF >= 0)
            def _():
                out_done(h - NBUF, hb)

            @pl.when(h < n_chunks)
            def _():
                start_gather(h, hb)

            _scale_chunk(bufs[b])
            start_out(c, b)
        return carry

    lax.fori_loop(0, n_groups, group_body, 0)
    for c in range(n_chunks - (NBUF - LOOKAHEAD), n_chunks):
        out_done(c, c % NBUF)


@functools.partial(jax.jit, static_argnames=())
def _emb_lookup(ids, W):
    batch, seq = ids.shape
    mesh = plsc.VectorSubcoreMesh(core_axis_name="c", subcore_axis_name="s")
    body = functools.partial(_emb_body, batch, seq)
    scratch = [pltpu.VMEM(((batch * seq) // NW,), jnp.int32)]
    scratch += [pltpu.VMEM((CHUNK, D_MODEL), jnp.float32) for _ in range(NBUF)]
    scratch += [pltpu.SemaphoreType.DMA for _ in range(2 * NBUF)]
    run = pl.kernel(
        body,
        out_type=jax.ShapeDtypeStruct((batch, seq, D_MODEL), jnp.float32),
        mesh=mesh,
        scratch_types=scratch,
    )
    return run(ids, W)


def kernel(input_ids, W):
    return _emb_lookup(input_ids, W)
